# Initial kernel scaffold; baseline (speedup 1.0000x reference)
import jax
import jax.numpy as jnp
from jax.experimental import pallas as pl


def kernel(graph_embedding, edge_index, e_feat, weight):
    raise NotImplementedError("write your pallas kernel here")


# speedup vs baseline: 7.0033x; 7.0033x over previous
import functools

import jax
import jax.numpy as jnp
from jax import lax
from jax.experimental import pallas as pl
from jax.experimental.pallas import tpu as pltpu
from jax.experimental.pallas import tpu_sc as plsc

N_NODES = 10000
N_EDGES = 320000
D = 128
ROWS = N_EDGES // D
ROWS_PER_CORE = ROWS // 2
NS = 16
N_PAD = 10240
NODE_ROWS_PER_TILE = N_PAD // NS
E_PAD = 10240
E_CHUNK = E_PAD // NS


def _sc_partials(g, src2d, dst2d, e2d, zrows, zeros_e):
    mesh = plsc.VectorSubcoreMesh(core_axis_name="c", subcore_axis_name="s")

    @functools.partial(
        pl.kernel,
        mesh=mesh,
        out_type=[
            jax.ShapeDtypeStruct((2, N_PAD, D), jnp.float32),
            jax.ShapeDtypeStruct((2, E_PAD), jnp.float32),
        ],
        scratch_types=[
            pltpu.VMEM_SHARED((N_PAD, D), jnp.float32),
            pltpu.VMEM_SHARED((E_PAD,), jnp.float32),
            pltpu.VMEM((D,), jnp.int32),
            pltpu.VMEM((D,), jnp.int32),
            pltpu.VMEM((D,), jnp.float32),
            pltpu.VMEM((D, D), jnp.float32),
            pltpu.SemaphoreType.DMA,
        ],
    )
    def k(g_hbm, src_hbm, dst_hbm, e_hbm, z_hbm, ze_hbm,
          part_hbm, parte_hbm,
          acc_sh, acce_sh, idx_s, idx_d, ev, rows, sem):
        c = lax.axis_index("c")
        s = lax.axis_index("s")

        pltpu.sync_copy(
            z_hbm, acc_sh.at[pl.ds(s * NODE_ROWS_PER_TILE, NODE_ROWS_PER_TILE)])
        pltpu.sync_copy(ze_hbm, acce_sh.at[pl.ds(s * E_CHUNK, E_CHUNK)])
        plsc.subcore_barrier()

        def body(kk, carry):
            r = s + NS * kk

            @pl.when(r < ROWS_PER_CORE)
            def _():
                row = c * ROWS_PER_CORE + r
                pltpu.sync_copy(src_hbm.at[row], idx_s)
                pltpu.sync_copy(dst_hbm.at[row], idx_d)
                pltpu.sync_copy(e_hbm.at[row], ev)
                pltpu.async_copy(g_hbm.at[idx_s], rows, sem).wait()
                pltpu.sync_copy(rows, acc_sh.at[idx_d], add=True)
                pltpu.sync_copy(ev, acce_sh.at[idx_d], add=True)
            return carry

        lax.fori_loop(0, (ROWS_PER_CORE + NS - 1) // NS, body, 0)
        plsc.subcore_barrier()

        pltpu.sync_copy(
            acc_sh.at[pl.ds(s * NODE_ROWS_PER_TILE, NODE_ROWS_PER_TILE)],
            part_hbm.at[c, pl.ds(s * NODE_ROWS_PER_TILE, NODE_ROWS_PER_TILE)])
        pltpu.sync_copy(
            acce_sh.at[pl.ds(s * E_CHUNK, E_CHUNK)],
            parte_hbm.at[c, pl.ds(s * E_CHUNK, E_CHUNK)])

    return k(g, src2d, dst2d, e2d, zrows, zeros_e)


def _tc_combine(part, parte3d, weight):
    R = 2048
    grid = N_PAD // R
    ER = E_PAD // D // grid

    def body(part_ref, parte_ref, w_ref, out_ref, oute_ref):
        out_ref[...] = (part_ref[0] + part_ref[1]) * w_ref[...]
        oute_ref[...] = parte_ref[0] + parte_ref[1]

    return pl.pallas_call(
        body,
        grid=(grid,),
        in_specs=[
            pl.BlockSpec((2, R, D), lambda i: (0, i, 0)),
            pl.BlockSpec((2, ER, D), lambda i: (0, i, 0)),
            pl.BlockSpec((1, D), lambda i: (0, 0)),
        ],
        out_specs=[
            pl.BlockSpec((R, D), lambda i: (i, 0)),
            pl.BlockSpec((ER, D), lambda i: (i, 0)),
        ],
        out_shape=[
            jax.ShapeDtypeStruct((N_PAD, D), jnp.float32),
            jax.ShapeDtypeStruct((E_PAD // D, D), jnp.float32),
        ],
    )(part, parte3d, weight)


def kernel(graph_embedding, edge_index, e_feat, weight):
    src2d = edge_index[0].astype(jnp.int32).reshape(ROWS, D)
    dst2d = edge_index[1].astype(jnp.int32).reshape(ROWS, D)
    e2d = e_feat.astype(jnp.float32).reshape(ROWS, D)
    zrows = jnp.zeros((NODE_ROWS_PER_TILE, D), jnp.float32)
    zeros_e = jnp.zeros((E_CHUNK,), jnp.float32)
    part, parte = _sc_partials(graph_embedding, src2d, dst2d, e2d, zrows, zeros_e)
    out128, oute = _tc_combine(part, parte.reshape(2, E_PAD // D, D), weight)
    return jnp.concatenate(
        [out128[:N_NODES], oute.reshape(E_PAD)[:N_NODES, None]], axis=1)
